# Initial kernel scaffold; baseline (speedup 1.0000x reference)
import jax
import jax.numpy as jnp
from jax.experimental import pallas as pl


def kernel(x, router_w, router_b, expert_w, expert_b):
    raise NotImplementedError("write your pallas kernel here")


# speedup vs baseline: 1.5228x; 1.5228x over previous
import functools

import jax
import jax.numpy as jnp
from jax import lax
from jax.experimental import pallas as pl
from jax.experimental.pallas import tpu as pltpu
from jax.experimental.pallas import tpu_sc as plsc

_LANES = 16


def _router_body(rw_ref, rb_ref, x_ref, d_ref):
    rdiff = rw_ref[0:1, :] - rw_ref[1:2, :]
    bdiff = rb_ref[0] - rb_ref[1]
    d = jax.lax.dot_general(
        rdiff, x_ref[...], (((1,), (1,)), ((), ())),
        preferred_element_type=jnp.float32)
    d_ref[0] = d + bdiff


def _router_d(xf, router_w, router_b, n_blk):
    bs, h = xf.shape
    blk = bs // n_blk
    d3 = pl.pallas_call(
        _router_body,
        grid=(n_blk,),
        in_specs=[
            pl.BlockSpec((router_w.shape[0], h), lambda i: (0, 0)),
            pl.BlockSpec(memory_space=pltpu.SMEM),
            pl.BlockSpec((blk, h), lambda i: (i, 0)),
        ],
        out_specs=pl.BlockSpec((1, 1, blk), lambda i: (i, 0, 0)),
        out_shape=jax.ShapeDtypeStruct((n_blk, 1, blk), jnp.float32),
    )(router_w, router_b, xf)
    return d3.reshape(bs)


def _make_sc_select(bs, h):
    mesh = plsc.VectorSubcoreMesh(core_axis_name="c", subcore_axis_name="s")
    n_chunks = bs // _LANES

    @functools.partial(
        pl.kernel,
        mesh=mesh,
        out_type=(
            jax.ShapeDtypeStruct((_LANES,), jnp.int32),
            jax.ShapeDtypeStruct((_LANES,), jnp.float32),
            jax.ShapeDtypeStruct((_LANES, h), jnp.float32),
        ),
        scratch_types=[
            pltpu.VMEM((bs,), jnp.float32),
            pltpu.VMEM((_LANES,), jnp.int32),
            pltpu.VMEM((_LANES,), jnp.float32),
            pltpu.VMEM((_LANES, h), jnp.float32),
            pltpu.VMEM((2 * _LANES,), jnp.float32),
            pltpu.VMEM((2 * _LANES,), jnp.int32),
            pltpu.SemaphoreType.DMA,
        ],
    )
    def sc_select(d_hbm, x_hbm, tok_out, gate_out, rows_out,
                  d_v, tok_v, gate_v, rows_v, buff, bufi, sem):
        cid = lax.axis_index("c")
        sid = lax.axis_index("s")

        @pl.when(jnp.logical_and(cid == 0, sid == 0))
        def _():
            pltpu.sync_copy(d_hbm, d_v)
            lane = lax.iota(jnp.int32, _LANES)
            neg = jnp.float32(-3.0e38)
            pos = jnp.float32(3.0e38)

            def xreduce(v, i, is_max):
                for sh in (1, 2, 4, 8):
                    buff[pl.ds(0, _LANES)] = v
                    buff[pl.ds(_LANES, _LANES)] = v
                    bufi[pl.ds(0, _LANES)] = i
                    bufi[pl.ds(_LANES, _LANES)] = i
                    v2 = buff[pl.ds(sh, _LANES)]
                    i2 = bufi[pl.ds(sh, _LANES)]
                    if is_max:
                        t = (v2 > v) | ((v2 == v) & (i2 < i))
                    else:
                        t = (v2 < v) | ((v2 == v) & (i2 < i))
                    v = jnp.where(t, v2, v)
                    i = jnp.where(t, i2, i)
                return v, i

            def xargmax(v, i):
                return xreduce(v, i, True)

            def xargmin(v, i):
                return xreduce(v, i, False)

            def pass1(i, carry):
                mx, imx, mn, imn = carry
                v = d_v[pl.ds(i * _LANES, _LANES)]
                idx = lane + i * _LANES
                gt = v > mx
                lt = v < mn
                return (jnp.where(gt, v, mx), jnp.where(gt, idx, imx),
                        jnp.where(lt, v, mn), jnp.where(lt, idx, imn))

            zi = jnp.zeros((_LANES,), jnp.int32)
            mx, imx, mn, imn = lax.fori_loop(
                0, n_chunks, pass1,
                (jnp.full((_LANES,), neg), zi, jnp.full((_LANES,), pos), zi))
            m1, i1 = xargmax(mx, imx)
            n1, j1 = xargmin(mn, imn)

            def pass2(i, carry):
                mx, imx, mn, imn = carry
                v = d_v[pl.ds(i * _LANES, _LANES)]
                idx = lane + i * _LANES
                vx = jnp.where(idx == i1, neg, v)
                vn = jnp.where(idx == j1, pos, v)
                gt = vx > mx
                lt = vn < mn
                return (jnp.where(gt, vx, mx), jnp.where(gt, idx, imx),
                        jnp.where(lt, vn, mn), jnp.where(lt, idx, imn))

            mx, imx, mn, imn = lax.fori_loop(
                0, n_chunks, pass2,
                (jnp.full((_LANES,), neg), zi, jnp.full((_LANES,), pos), zi))
            m2, i2 = xargmax(mx, imx)
            n2, j2 = xargmin(mn, imn)

            z = jnp.where(lane == 0, m1,
                          jnp.where(lane == 1, -n1,
                                    jnp.where(lane == 2, m2,
                                              jnp.where(lane == 3, -n2, 0.0))))
            gates = 1.0 / (1.0 + jnp.exp(-z))
            gates = jnp.where(lane < 4, gates, 0.0)
            toks = jnp.where(lane == 0, i1,
                             jnp.where(lane == 1, i2,
                                       jnp.where(lane == 2, j1,
                                                 jnp.where(lane == 3, j2, 0))))
            tok_v[...] = toks
            gate_v[...] = gates
            pltpu.sync_copy(tok_v, tok_out)
            pltpu.sync_copy(gate_v, gate_out)
            pltpu.async_copy(x_hbm.at[tok_v], rows_v, sem).wait()
            pltpu.sync_copy(rows_v, rows_out)

    return sc_select


def _out_body(tok_ref, gate_ref, rows_ref, w_ref, b_ref, y_ref, eout_ref):
    j = pl.program_id(0)
    blk = y_ref.shape[0]

    @pl.when(j == 0)
    def _():
        eout_ref[...] = jax.lax.dot_general(
            rows_ref[...], w_ref[...], (((1,), (1,)), ((), ())),
            preferred_element_type=jnp.float32) + b_ref[...]

    y_ref[...] = jnp.zeros(y_ref.shape, y_ref.dtype)
    for t in range(4):
        tok = tok_ref[t]
        g = gate_ref[t]
        loc = tok % blk

        @pl.when(tok // blk == j)
        def _():
            y_ref[pl.ds(loc, 1), :] = (y_ref[pl.ds(loc, 1), :]
                                       + g * eout_ref[pl.ds(t, 1), :])


def _expert_scatter(toks, gates, rows, expert_w, expert_b, bs, n_blk):
    h = expert_w.shape[0]
    blk = bs // n_blk
    return pl.pallas_call(
        _out_body,
        grid=(n_blk,),
        in_specs=[
            pl.BlockSpec(memory_space=pltpu.SMEM),
            pl.BlockSpec(memory_space=pltpu.SMEM),
            pl.BlockSpec((_LANES, h), lambda j: (0, 0)),
            pl.BlockSpec((h, h), lambda j: (0, 0)),
            pl.BlockSpec((1, h), lambda j: (0, 0)),
        ],
        out_specs=pl.BlockSpec((blk, h), lambda j: (j, 0)),
        out_shape=jax.ShapeDtypeStruct((bs, h), jnp.float32),
        scratch_shapes=[pltpu.VMEM((_LANES, h), jnp.float32)],
    )(toks, gates, rows, expert_w, expert_b.reshape(1, h))


def kernel(x, router_w, router_b, expert_w, expert_b):
    b, s, h = x.shape
    xf = x.reshape(-1, h)
    bs = xf.shape[0]
    n_blk = 8
    d = _router_d(xf, router_w, router_b, n_blk)
    toks, gates, rows = _make_sc_select(bs, h)(d, xf)
    y = _expert_scatter(toks, gates, rows, expert_w, expert_b, bs, n_blk)
    return y.reshape(b, s, h)
